# baseline (speedup 1.0000x reference)
import functools

import jax
import jax.numpy as jnp
from jax import lax
from jax.experimental import pallas as pl
from jax.experimental.pallas import tpu as pltpu
from jax.experimental.pallas import tpu_sc as plsc

_VOCAB = 100000
_DIM = 128
_BATCH = 4096
_SEQ = 200
_NCLS = 2

_TC_BLK = 25088
_TC_GRID = -(-_VOCAB // _TC_BLK)
_VOCAB_PAD = _TC_GRID * _TC_BLK


def _tc_project_body(t_ref, w_ref, b_ref, o_ref):
    res = (
        lax.dot_general(
            w_ref[...],
            t_ref[...],
            (((0,), (1,)), ((), ())),
            preferred_element_type=jnp.float32,
        )
        + b_ref[...]
    )
    halves = lax.bitcast_convert_type(
        res.astype(jnp.bfloat16), jnp.uint16
    ).astype(jnp.uint32)
    packed = (halves[1:2, :] << 16) | halves[0:1, :]
    o_ref[...] = lax.bitcast_convert_type(packed, jnp.int32)


def _tc_project(table, W, b):
    return pl.pallas_call(
        _tc_project_body,
        grid=(_TC_GRID,),
        in_specs=[
            pl.BlockSpec((_TC_BLK, _DIM), lambda i: (i, 0)),
            pl.BlockSpec((_DIM, _NCLS), lambda i: (0, 0)),
            pl.BlockSpec((_NCLS, 1), lambda i: (0, 0)),
        ],
        out_specs=pl.BlockSpec((1, _TC_BLK), lambda i: (0, i)),
        out_shape=jax.ShapeDtypeStruct((1, _VOCAB_PAD), jnp.int32),
    )(table, W, b.reshape(_NCLS, 1))


_VOCAB_STAGE = 100096
_N_TILES = 32
_ROWS_PER_TILE = _BATCH // _N_TILES
_N_GROUPS = _ROWS_PER_TILE // 16
_CHUNK_T = 50
_N_CHUNKS = _SEQ // _CHUNK_T


@functools.partial(
    pl.kernel,
    out_type=jax.ShapeDtypeStruct((_NCLS, _BATCH), jnp.float32),
    mesh=plsc.VectorSubcoreMesh(core_axis_name="c", subcore_axis_name="s"),
    compiler_params=pltpu.CompilerParams(needs_layout_passes=False),
    scratch_types=[
        pltpu.VMEM((_VOCAB_STAGE,), jnp.int32),
        pltpu.VMEM_SHARED((_VOCAB_STAGE,), jnp.int32),
        pltpu.VMEM((_CHUNK_T * _ROWS_PER_TILE,), jnp.int32),
        pltpu.VMEM((_CHUNK_T * _ROWS_PER_TILE,), jnp.int32),
        pltpu.VMEM((_ROWS_PER_TILE,), jnp.float32),
        pltpu.VMEM((_ROWS_PER_TILE,), jnp.float32),
        pltpu.SemaphoreType.DMA,
        pltpu.SemaphoreType.DMA,
        pltpu.SemaphoreType.DMA,
    ],
)
def _sc_pool(
    pt_hbm, texts_hbm, out_hbm, pcol, spcol, tka, tkb, ob0, ob1, semp, sema, semb
):
    sub = lax.axis_index("s")
    wid = sub * 2 + lax.axis_index("c")
    r0 = wid * _ROWS_PER_TILE

    bufs = (tka, tkb)
    sems = (sema, semb)
    csz = _CHUNK_T * _ROWS_PER_TILE

    def chunk_src(ct):
        return texts_hbm.at[wid, pl.ds(ct * csz, csz)]

    pending = [None] * _N_CHUNKS
    pending[0] = pltpu.async_copy(chunk_src(0), bufs[0], sems[0])

    @pl.when(sub == 0)
    def _():
        pltpu.sync_copy(pt_hbm.at[0, pl.ds(0, _VOCAB_STAGE)], spcol)

    plsc.subcore_barrier()
    pltpu.async_copy(spcol, pcol, semp).wait()

    inv_l = jnp.float32(1.0 / _SEQ)
    zero = jnp.zeros((16,), jnp.float32)
    himask = jnp.full((16,), -65536, jnp.int32)

    accs = (zero,) * (2 * _N_GROUPS)
    for ct in range(_N_CHUNKS):
        if ct + 1 < _N_CHUNKS:
            pending[ct + 1] = pltpu.async_copy(
                chunk_src(ct + 1), bufs[(ct + 1) % 2], sems[(ct + 1) % 2]
            )
        pending[ct].wait()
        tbuf = bufs[ct % 2]

        def body(t, accs):
            out = []
            for g in range(_N_GROUPS):
                tok = tbuf[pl.ds(t * _ROWS_PER_TILE + g * 16, 16)]
                val = plsc.load_gather(pcol, [tok])
                c0 = plsc.bitcast(val << 16, jnp.float32)
                c1 = plsc.bitcast(val & himask, jnp.float32)
                out.append(accs[2 * g] + c0)
                out.append(accs[2 * g + 1] + c1)
            return tuple(out)

        accs = lax.fori_loop(0, _CHUNK_T, body, accs, unroll=4)

    for g in range(_N_GROUPS):
        ob0[pl.ds(g * 16, 16)] = accs[2 * g] * inv_l
        ob1[pl.ds(g * 16, 16)] = accs[2 * g + 1] * inv_l

    pltpu.sync_copy(ob0, out_hbm.at[0, pl.ds(r0, _ROWS_PER_TILE)])
    pltpu.sync_copy(ob1, out_hbm.at[1, pl.ds(r0, _ROWS_PER_TILE)])


def kernel(texts, table, W, b):
    texts_t = (
        texts.astype(jnp.int32)
        .T.reshape(_SEQ, _N_TILES, _ROWS_PER_TILE)
        .transpose(1, 0, 2)
        .reshape(_N_TILES, _SEQ * _ROWS_PER_TILE)
    )
    pt = _tc_project(table, W, b)
    out_t = _sc_pool(pt, texts_t)
    return out_t.T


# speedup vs baseline: 1.0120x; 1.0120x over previous
import functools

import jax
import jax.numpy as jnp
from jax import lax
from jax.experimental import pallas as pl
from jax.experimental.pallas import tpu as pltpu
from jax.experimental.pallas import tpu_sc as plsc

_VOCAB = 100000
_DIM = 128
_BATCH = 4096
_SEQ = 200
_NCLS = 2

_TC_BLK = 25088
_TC_GRID = -(-_VOCAB // _TC_BLK)
_VOCAB_PAD = _TC_GRID * _TC_BLK


def _tc_project_body(t_ref, w_ref, b_ref, o_ref):
    res = (
        lax.dot_general(
            w_ref[...],
            t_ref[...],
            (((0,), (1,)), ((), ())),
            preferred_element_type=jnp.float32,
        )
        + b_ref[...]
    )
    halves = lax.bitcast_convert_type(
        res.astype(jnp.bfloat16), jnp.uint16
    ).astype(jnp.uint32)
    packed = (halves[1:2, :] << 16) | halves[0:1, :]
    o_ref[...] = lax.bitcast_convert_type(packed, jnp.int32)


def _tc_project(table, W, b):
    return pl.pallas_call(
        _tc_project_body,
        grid=(_TC_GRID,),
        in_specs=[
            pl.BlockSpec((_TC_BLK, _DIM), lambda i: (i, 0)),
            pl.BlockSpec((_DIM, _NCLS), lambda i: (0, 0)),
            pl.BlockSpec((_NCLS, 1), lambda i: (0, 0)),
        ],
        out_specs=pl.BlockSpec((1, _TC_BLK), lambda i: (0, i)),
        out_shape=jax.ShapeDtypeStruct((1, _VOCAB_PAD), jnp.int32),
    )(table, W, b.reshape(_NCLS, 1))


_VOCAB_STAGE = 100096
_N_TILES = 32
_ROWS_PER_TILE = _BATCH // _N_TILES
_N_GROUPS = _ROWS_PER_TILE // 16
_CHUNK_T = 50
_N_CHUNKS = _SEQ // _CHUNK_T


@functools.partial(
    pl.kernel,
    out_type=jax.ShapeDtypeStruct((_NCLS, _BATCH), jnp.float32),
    mesh=plsc.VectorSubcoreMesh(core_axis_name="c", subcore_axis_name="s"),
    compiler_params=pltpu.CompilerParams(needs_layout_passes=False),
    scratch_types=[
        pltpu.VMEM((_VOCAB_STAGE,), jnp.int32),
        pltpu.VMEM_SHARED((_VOCAB_STAGE,), jnp.int32),
        pltpu.VMEM((_CHUNK_T * _ROWS_PER_TILE,), jnp.int32),
        pltpu.VMEM((_CHUNK_T * _ROWS_PER_TILE,), jnp.int32),
        pltpu.VMEM((_ROWS_PER_TILE,), jnp.float32),
        pltpu.VMEM((_ROWS_PER_TILE,), jnp.float32),
        pltpu.SemaphoreType.DMA,
        pltpu.SemaphoreType.DMA,
        pltpu.SemaphoreType.DMA,
    ],
)
def _sc_pool(
    pt_hbm, texts_hbm, out_hbm, pcol, spcol, tka, tkb, ob0, ob1, semp, sema, semb
):
    sub = lax.axis_index("s")
    wid = sub * 2 + lax.axis_index("c")
    r0 = wid * _ROWS_PER_TILE

    bufs = (tka, tkb)
    sems = (sema, semb)
    csz = _CHUNK_T * _ROWS_PER_TILE

    def chunk_src(ct):
        return texts_hbm.at[wid, pl.ds(ct * csz, csz)]

    pending = [None] * _N_CHUNKS
    pending[0] = pltpu.async_copy(chunk_src(0), bufs[0], sems[0])

    @pl.when(sub == 0)
    def _():
        pltpu.sync_copy(pt_hbm.at[0, pl.ds(0, _VOCAB_STAGE)], spcol)

    plsc.subcore_barrier()
    pltpu.async_copy(spcol, pcol, semp).wait()

    inv_l = jnp.float32(1.0 / _SEQ)
    zero = jnp.zeros((16,), jnp.float32)
    himask = jnp.full((16,), -65536, jnp.int32)

    accs = (zero,) * (2 * _N_GROUPS)
    for ct in range(_N_CHUNKS):
        if ct + 1 < _N_CHUNKS:
            pending[ct + 1] = pltpu.async_copy(
                chunk_src(ct + 1), bufs[(ct + 1) % 2], sems[(ct + 1) % 2]
            )
        pending[ct].wait()
        tbuf = bufs[ct % 2]

        def body(t, accs):
            out = []
            for g in range(_N_GROUPS):
                tok = tbuf[pl.ds(t * _ROWS_PER_TILE + g * 16, 16)]
                val = plsc.load_gather(pcol, [tok])
                c0 = plsc.bitcast(val << 16, jnp.float32)
                c1 = plsc.bitcast(val & himask, jnp.float32)
                out.append(accs[2 * g] + c0)
                out.append(accs[2 * g + 1] + c1)
            return tuple(out)

        accs = lax.fori_loop(0, _CHUNK_T, body, accs, unroll=2)

    for g in range(_N_GROUPS):
        ob0[pl.ds(g * 16, 16)] = accs[2 * g] * inv_l
        ob1[pl.ds(g * 16, 16)] = accs[2 * g + 1] * inv_l

    pltpu.sync_copy(ob0, out_hbm.at[0, pl.ds(r0, _ROWS_PER_TILE)])
    pltpu.sync_copy(ob1, out_hbm.at[1, pl.ds(r0, _ROWS_PER_TILE)])


def kernel(texts, table, W, b):
    texts_t = (
        texts.astype(jnp.int32)
        .T.reshape(_SEQ, _N_TILES, _ROWS_PER_TILE)
        .transpose(1, 0, 2)
        .reshape(_N_TILES, _SEQ * _ROWS_PER_TILE)
    )
    pt = _tc_project(table, W, b)
    out_t = _sc_pool(pt, texts_t)
    return out_t.T
